# baseline (speedup 1.0000x reference)
import functools

import jax
import jax.numpy as jnp
from jax import lax
from jax.experimental import pallas as pl
from jax.experimental.pallas import tpu as pltpu
from jax.experimental.pallas import tpu_sc as plsc

NC = 2
NS = 16
LANES = 16

N_PAD = 10240
E_PAD = 163840
EB = 128
NB = E_PAD // (NS * EB)
DH = 512
DC = 128
NCHUNK = DH // DC
RPT = N_PAD // NS
BN = 256


def _sc_mesh():
    return plsc.VectorSubcoreMesh(
        core_axis_name="c", subcore_axis_name="s", num_cores=NC, num_subcores=NS
    )


def _fill_rows(buf, nrows, width, vec):
    for r in range(nrows):
        for i in range(width // LANES):
            buf[r, pl.ds(i * LANES, LANES)] = vec


def _zero_accum_slice(zbuf, zrows, accum, base, nrows):
    off = 0
    while off < nrows:
        step = min(zrows, nrows - off)
        if step == zrows:
            pltpu.sync_copy(zbuf, accum.at[pl.ds(base + off, step)])
        else:
            pltpu.sync_copy(zbuf.at[pl.ds(0, step)], accum.at[pl.ds(base + off, step)])
        off += step


def _deg_body(col_hbm, out_hbm, colbuf, onesbuf, zbuf, accum):
    c = lax.axis_index("c")
    s = lax.axis_index("s")
    _fill_rows(zbuf, 64, LANES, jnp.zeros((LANES,), jnp.float32))
    _fill_rows(onesbuf, EB, LANES, jnp.ones((LANES,), jnp.float32))
    _zero_accum_slice(zbuf, 64, accum, s * RPT, RPT)
    pltpu.sync_copy(col_hbm.at[s].at[pl.ds(c * (NB // NC), NB // NC)], colbuf)
    plsc.subcore_barrier()

    @pl.loop(0, NB // NC)
    def _(j):
        pltpu.sync_copy(onesbuf, accum.at[colbuf.at[j]], add=True)

    plsc.subcore_barrier()
    pltpu.sync_copy(
        accum.at[pl.ds(s * RPT, RPT)], out_hbm.at[c].at[pl.ds(s * RPT, RPT)]
    )


def _deg_counts(col3):
    k = pl.kernel(
        _deg_body,
        out_type=jax.ShapeDtypeStruct((NC, N_PAD, LANES), jnp.float32),
        mesh=_sc_mesh(),
        scratch_types=[
            pltpu.VMEM((NB // NC, EB), jnp.int32),
            pltpu.VMEM((EB, LANES), jnp.float32),
            pltpu.VMEM((64, LANES), jnp.float32),
            pltpu.VMEM_SHARED((N_PAD, LANES), jnp.float32),
        ],
    )
    return k(col3)


NBUF = 2
NGRP = NB // NBUF


def _scatter_body(g_hbm, rc_hbm, out_hbm, ibuf, gbuf, zbuf, accum, *sems):
    gsem = sems[0:NBUF]
    ssem = sems[NBUF:2 * NBUF]
    isem = sems[2 * NBUF:2 * NBUF + 2]
    c = lax.axis_index("c")
    s = lax.axis_index("s")
    _fill_rows(zbuf, 16, DC, jnp.zeros((LANES,), jnp.float32))

    def i_desc(par, grp):
        return pltpu.make_async_copy(rc_hbm.at[s].at[grp], ibuf.at[par], isem[par])

    def g_desc(b, par, chunk):
        return pltpu.make_async_copy(
            g_hbm.at[chunk].at[ibuf.at[par, b, 0]], gbuf.at[b], gsem[b]
        )

    def s_desc(b, par):
        return pltpu.make_async_copy(
            gbuf.at[b], accum.at[ibuf.at[par, b, 1]], ssem[b]
        )

    for kk in range(NCHUNK // NC):
        chunk = c * (NCHUNK // NC) + kk
        zd = [
            pltpu.make_async_copy(
                zbuf, accum.at[pl.ds(s * RPT + 16 * t, 16)], gsem[0]
            )
            for t in range(RPT // 16)
        ]
        for d in zd:
            d.start()
        for d in zd:
            d.wait()
        plsc.subcore_barrier()

        def group_tail(par, nxt_grp, prefetch_grp, chunk):
            nxt = 1 - par
            i_desc(nxt, nxt_grp).wait()
            for b in range(NBUF):
                s_desc(b, par).wait()
                g_desc(b, nxt, chunk).start()
            if prefetch_grp is not None:
                i_desc(par, prefetch_grp).start()

        def group_head(par, chunk):
            for b in range(NBUF):
                g_desc(b, par, chunk).wait()
                s_desc(b, par).start(add=True)

        @pl.loop(0, NGRP)
        def _(grp):
            i_desc(0, grp).start()
            i_desc(0, grp).wait()
            for b in range(NBUF):
                g_desc(b, 0, chunk).start()
            for b in range(NBUF):
                g_desc(b, 0, chunk).wait()
            for b in range(NBUF):
                s_desc(b, 0).start(add=True)
            for b in range(NBUF):
                s_desc(b, 0).wait()

        plsc.subcore_barrier()
        pltpu.sync_copy(
            accum.at[pl.ds(s * RPT, RPT)],
            out_hbm.at[chunk].at[pl.ds(s * RPT, RPT)],
        )
        plsc.subcore_barrier()


def _edge_scatter(g, rc):
    k = pl.kernel(
        _scatter_body,
        out_type=jax.ShapeDtypeStruct((NCHUNK, N_PAD, DC), jnp.float32),
        mesh=_sc_mesh(),
        scratch_types=[
            pltpu.VMEM((2, NBUF, 2, EB), jnp.int32),
            pltpu.VMEM((NBUF, EB, DC), jnp.float32),
            pltpu.VMEM((16, DC), jnp.float32),
            pltpu.VMEM_SHARED((N_PAD, DC), jnp.float32),
        ] + [pltpu.SemaphoreType.DMA] * (2 * NBUF + 2),
    )
    return k(g, rc)


def _dinv(degp_ref):
    p = degp_ref[...]
    cnt = p[0, :, 0] + p[1, :, 0]
    return lax.rsqrt(cnt + 1.0)


def _m1_body(degp_ref, x_ref, w_ref, b_ref, g_ref, u_ref):
    dinv = _dinv(degp_ref)
    h = (
        jnp.dot(x_ref[...], w_ref[...], preferred_element_type=jnp.float32)
        + b_ref[...]
    )
    g_ref[0] = dinv[:, None] * h
    u_ref[...] = (dinv * dinv)[:, None] * h


def _m1(degp, x_p, w_t, b_r):
    d_in = x_p.shape[1]
    return pl.pallas_call(
        _m1_body,
        grid=(N_PAD // BN, DH // DC),
        in_specs=[
            pl.BlockSpec((NC, BN, LANES), lambda i, j: (0, i, 0)),
            pl.BlockSpec((BN, d_in), lambda i, j: (i, 0)),
            pl.BlockSpec((d_in, DC), lambda i, j: (0, j)),
            pl.BlockSpec((1, DC), lambda i, j: (0, j)),
        ],
        out_specs=[
            pl.BlockSpec((1, BN, DC), lambda i, j: (j, i, 0)),
            pl.BlockSpec((BN, DC), lambda i, j: (i, j)),
        ],
        out_shape=[
            jax.ShapeDtypeStruct((NCHUNK, N_PAD, DC), jnp.float32),
            jax.ShapeDtypeStruct((N_PAD, DH), jnp.float32),
        ],
    )(degp, x_p, w_t, b_r)


def _m2_body(degp_ref, s_ref, u_ref, w_ref, b_ref, g_ref, u2_ref, acc):
    k = pl.program_id(2)
    dinv = _dinv(degp_ref)
    d2 = dinv * dinv
    z = jnp.maximum(dinv[:, None] * s_ref[0] + d2[:, None] * u_ref[...], 0.0)

    @pl.when(k == 0)
    def _():
        acc[...] = jnp.zeros_like(acc)

    acc[...] += jnp.dot(z, w_ref[...], preferred_element_type=jnp.float32)

    @pl.when(k == NCHUNK - 1)
    def _():
        h = acc[...] + b_ref[...]
        g_ref[0] = dinv[:, None] * h
        u2_ref[...] = d2[:, None] * h


def _m2(degp, s1, u1, w_t, b_r):
    return pl.pallas_call(
        _m2_body,
        grid=(N_PAD // BN, DH // DC, NCHUNK),
        in_specs=[
            pl.BlockSpec((NC, BN, LANES), lambda i, j, k: (0, i, 0)),
            pl.BlockSpec((1, BN, DC), lambda i, j, k: (k, i, 0)),
            pl.BlockSpec((BN, DC), lambda i, j, k: (i, k)),
            pl.BlockSpec((DC, DC), lambda i, j, k: (k, j)),
            pl.BlockSpec((1, DC), lambda i, j, k: (0, j)),
        ],
        out_specs=[
            pl.BlockSpec((1, BN, DC), lambda i, j, k: (j, i, 0)),
            pl.BlockSpec((BN, DC), lambda i, j, k: (i, j)),
        ],
        out_shape=[
            jax.ShapeDtypeStruct((NCHUNK, N_PAD, DC), jnp.float32),
            jax.ShapeDtypeStruct((N_PAD, DH), jnp.float32),
        ],
        scratch_shapes=[pltpu.VMEM((BN, DC), jnp.float32)],
    )(degp, s1, u1, w_t, b_r)


def _m3_body(degp_ref, s_ref, u_ref, w_ref, b_ref, o_ref, acc):
    k = pl.program_id(1)
    dinv = _dinv(degp_ref)
    d2 = dinv * dinv
    z = jnp.maximum(dinv[:, None] * s_ref[0] + d2[:, None] * u_ref[...], 0.0)

    @pl.when(k == 0)
    def _():
        acc[...] = jnp.zeros_like(acc)

    acc[...] += jnp.dot(z, w_ref[...], preferred_element_type=jnp.float32)

    @pl.when(k == NCHUNK - 1)
    def _():
        o_ref[...] = acc[...] + b_ref[...]


def _m3(degp, s2, u2, w_t, b_r):
    d_out = w_t.shape[1]
    return pl.pallas_call(
        _m3_body,
        grid=(N_PAD // BN, NCHUNK),
        in_specs=[
            pl.BlockSpec((NC, BN, LANES), lambda i, k: (0, i, 0)),
            pl.BlockSpec((1, BN, DC), lambda i, k: (k, i, 0)),
            pl.BlockSpec((BN, DC), lambda i, k: (i, k)),
            pl.BlockSpec((DC, d_out), lambda i, k: (k, 0)),
            pl.BlockSpec((1, d_out), lambda i, k: (0, 0)),
        ],
        out_specs=pl.BlockSpec((BN, d_out), lambda i, k: (i, 0)),
        out_shape=jax.ShapeDtypeStruct((N_PAD, d_out), jnp.float32),
        scratch_shapes=[pltpu.VMEM((BN, d_out), jnp.float32)],
    )(degp, s2, u2, w_t, b_r)


def kernel(x, edge_index, W1, b1, W2, b2, W3, b3):
    n, _ = x.shape
    e = edge_index.shape[1]
    x_p = jnp.pad(x, ((0, N_PAD - n), (0, 0)))
    row3 = jnp.pad(edge_index[0], (0, E_PAD - e)).reshape(NS, NB, EB)
    col3 = jnp.pad(edge_index[1], (0, E_PAD - e), constant_values=n).reshape(
        NS, NB, EB
    )
    rc = jnp.stack(
        [row3.reshape(NS, NGRP, NBUF, EB), col3.reshape(NS, NGRP, NBUF, EB)],
        axis=3,
    )

    degp = _deg_counts(col3)
    g1, u1 = _m1(degp, x_p, W1.T, b1.reshape(1, -1))
    s1 = _edge_scatter(g1, rc)
    g2, u2 = _m2(degp, s1, u1, W2.T, b2.reshape(1, -1))
    s2 = _edge_scatter(g2, rc)
    y = _m3(degp, s2, u2, W3.T, b3.reshape(1, -1))
    return y[:n]


# speedup vs baseline: 1.0636x; 1.0636x over previous
import functools

import jax
import jax.numpy as jnp
from jax import lax
from jax.experimental import pallas as pl
from jax.experimental.pallas import tpu as pltpu
from jax.experimental.pallas import tpu_sc as plsc

NC = 2
NS = 16
LANES = 16

N_PAD = 10240
E_PAD = 163840
EB = 128
NB = E_PAD // (NS * EB)
DH = 512
DC = 128
NCHUNK = DH // DC
RPT = N_PAD // NS
BN = 256


def _sc_mesh():
    return plsc.VectorSubcoreMesh(
        core_axis_name="c", subcore_axis_name="s", num_cores=NC, num_subcores=NS
    )


def _fill_rows(buf, nrows, width, vec):
    for r in range(nrows):
        for i in range(width // LANES):
            buf[r, pl.ds(i * LANES, LANES)] = vec


def _zero_accum_slice(zbuf, zrows, accum, base, nrows):
    off = 0
    while off < nrows:
        step = min(zrows, nrows - off)
        if step == zrows:
            pltpu.sync_copy(zbuf, accum.at[pl.ds(base + off, step)])
        else:
            pltpu.sync_copy(zbuf.at[pl.ds(0, step)], accum.at[pl.ds(base + off, step)])
        off += step


def _deg_body(col_hbm, out_hbm, colbuf, onesbuf, zbuf, accum):
    c = lax.axis_index("c")
    s = lax.axis_index("s")
    _fill_rows(zbuf, 64, LANES, jnp.zeros((LANES,), jnp.float32))
    _fill_rows(onesbuf, EB, LANES, jnp.ones((LANES,), jnp.float32))
    _zero_accum_slice(zbuf, 64, accum, s * RPT, RPT)
    pltpu.sync_copy(col_hbm.at[s].at[pl.ds(c * (NB // NC), NB // NC)], colbuf)
    plsc.subcore_barrier()

    @pl.loop(0, NB // NC)
    def _(j):
        pltpu.sync_copy(onesbuf, accum.at[colbuf.at[j]], add=True)

    plsc.subcore_barrier()
    pltpu.sync_copy(
        accum.at[pl.ds(s * RPT, RPT)], out_hbm.at[c].at[pl.ds(s * RPT, RPT)]
    )


def _deg_counts(col3):
    k = pl.kernel(
        _deg_body,
        out_type=jax.ShapeDtypeStruct((NC, N_PAD, LANES), jnp.float32),
        mesh=_sc_mesh(),
        scratch_types=[
            pltpu.VMEM((NB // NC, EB), jnp.int32),
            pltpu.VMEM((EB, LANES), jnp.float32),
            pltpu.VMEM((64, LANES), jnp.float32),
            pltpu.VMEM_SHARED((N_PAD, LANES), jnp.float32),
        ],
    )
    return k(col3)


SEB = 64
NBATCH = N_PAD // SEB
NBUF = 4
SGB = 32
NSG = NBATCH // SGB


def _scatter_body(g_hbm, rc_hbm, out_hbm, ibuf, gbuf, zbuf, accum, *sems):
    gsem = sems[0:NBUF]
    ssem = sems[NBUF:2 * NBUF]
    isem = sems[2 * NBUF]
    c = lax.axis_index("c")
    s = lax.axis_index("s")
    _fill_rows(zbuf, 8, DC, jnp.zeros((LANES,), jnp.float32))
    for kk in range(NCHUNK // NC):
        chunk = c * (NCHUNK // NC) + kk
        zd = [
            pltpu.make_async_copy(
                zbuf, accum.at[pl.ds(s * RPT + 8 * t, 8)], gsem[0]
            )
            for t in range(RPT // 8)
        ]
        for d in zd:
            d.start()
        for d in zd:
            d.wait()
        plsc.subcore_barrier()

        @pl.loop(0, NSG)
        def _(sg):
            idx = pltpu.make_async_copy(
                rc_hbm.at[s].at[pl.ds(sg * SGB, SGB)], ibuf, isem
            )
            idx.start()
            idx.wait()

            def gd(t):
                return pltpu.make_async_copy(
                    g_hbm.at[chunk].at[ibuf.at[t, 0]],
                    gbuf.at[t % NBUF],
                    gsem[t % NBUF],
                )

            def sd(t):
                return pltpu.make_async_copy(
                    gbuf.at[t % NBUF], accum.at[ibuf.at[t, 1]], ssem[t % NBUF]
                )

            gd(0).start()
            gd(1).start()
            for t in range(SGB):
                gd(t).wait()
                if t >= 2:
                    sd(t - 2).wait()
                sd(t).start(add=True)
                if t + 2 < SGB:
                    gd(t + 2).start()
            sd(SGB - 2).wait()
            sd(SGB - 1).wait()

        plsc.subcore_barrier()
        pltpu.sync_copy(
            accum.at[pl.ds(s * RPT, RPT)],
            out_hbm.at[chunk].at[pl.ds(s * RPT, RPT)],
        )
        plsc.subcore_barrier()


def _edge_scatter(g, rc):
    k = pl.kernel(
        _scatter_body,
        out_type=jax.ShapeDtypeStruct((NCHUNK, N_PAD, DC), jnp.float32),
        mesh=_sc_mesh(),
        scratch_types=[
            pltpu.VMEM((SGB, 2, SEB), jnp.int32),
            pltpu.VMEM((NBUF, SEB, DC), jnp.float32),
            pltpu.VMEM((8, DC), jnp.float32),
            pltpu.VMEM_SHARED((N_PAD, DC), jnp.float32),
        ] + [pltpu.SemaphoreType.DMA] * (2 * NBUF + 1),
    )
    return k(g, rc)


def _dinv(degp_ref):
    p = degp_ref[...]
    cnt = p[0, :, 0] + p[1, :, 0]
    return lax.rsqrt(cnt + 1.0)


def _m1_body(degp_ref, x_ref, w_ref, b_ref, g_ref, u_ref):
    dinv = _dinv(degp_ref)
    h = (
        jnp.dot(x_ref[...], w_ref[...], preferred_element_type=jnp.float32)
        + b_ref[...]
    )
    g_ref[0] = dinv[:, None] * h
    u_ref[...] = (dinv * dinv)[:, None] * h


def _m1(degp, x_p, w_t, b_r):
    d_in = x_p.shape[1]
    return pl.pallas_call(
        _m1_body,
        grid=(N_PAD // BN, DH // DC),
        in_specs=[
            pl.BlockSpec((NC, BN, LANES), lambda i, j: (0, i, 0)),
            pl.BlockSpec((BN, d_in), lambda i, j: (i, 0)),
            pl.BlockSpec((d_in, DC), lambda i, j: (0, j)),
            pl.BlockSpec((1, DC), lambda i, j: (0, j)),
        ],
        out_specs=[
            pl.BlockSpec((1, BN, DC), lambda i, j: (j, i, 0)),
            pl.BlockSpec((BN, DC), lambda i, j: (i, j)),
        ],
        out_shape=[
            jax.ShapeDtypeStruct((NCHUNK, N_PAD, DC), jnp.float32),
            jax.ShapeDtypeStruct((N_PAD, DH), jnp.float32),
        ],
    )(degp, x_p, w_t, b_r)


def _m2_body(degp_ref, s_ref, u_ref, w_ref, b_ref, g_ref, u2_ref, acc):
    k = pl.program_id(2)
    dinv = _dinv(degp_ref)
    d2 = dinv * dinv
    z = jnp.maximum(dinv[:, None] * s_ref[0] + d2[:, None] * u_ref[...], 0.0)

    @pl.when(k == 0)
    def _():
        acc[...] = jnp.zeros_like(acc)

    acc[...] += jnp.dot(z, w_ref[...], preferred_element_type=jnp.float32)

    @pl.when(k == NCHUNK - 1)
    def _():
        h = acc[...] + b_ref[...]
        g_ref[0] = dinv[:, None] * h
        u2_ref[...] = d2[:, None] * h


def _m2(degp, s1, u1, w_t, b_r):
    return pl.pallas_call(
        _m2_body,
        grid=(N_PAD // BN, DH // DC, NCHUNK),
        in_specs=[
            pl.BlockSpec((NC, BN, LANES), lambda i, j, k: (0, i, 0)),
            pl.BlockSpec((1, BN, DC), lambda i, j, k: (k, i, 0)),
            pl.BlockSpec((BN, DC), lambda i, j, k: (i, k)),
            pl.BlockSpec((DC, DC), lambda i, j, k: (k, j)),
            pl.BlockSpec((1, DC), lambda i, j, k: (0, j)),
        ],
        out_specs=[
            pl.BlockSpec((1, BN, DC), lambda i, j, k: (j, i, 0)),
            pl.BlockSpec((BN, DC), lambda i, j, k: (i, j)),
        ],
        out_shape=[
            jax.ShapeDtypeStruct((NCHUNK, N_PAD, DC), jnp.float32),
            jax.ShapeDtypeStruct((N_PAD, DH), jnp.float32),
        ],
        scratch_shapes=[pltpu.VMEM((BN, DC), jnp.float32)],
    )(degp, s1, u1, w_t, b_r)


def _m3_body(degp_ref, s_ref, u_ref, w_ref, b_ref, o_ref, acc):
    k = pl.program_id(1)
    dinv = _dinv(degp_ref)
    d2 = dinv * dinv
    z = jnp.maximum(dinv[:, None] * s_ref[0] + d2[:, None] * u_ref[...], 0.0)

    @pl.when(k == 0)
    def _():
        acc[...] = jnp.zeros_like(acc)

    acc[...] += jnp.dot(z, w_ref[...], preferred_element_type=jnp.float32)

    @pl.when(k == NCHUNK - 1)
    def _():
        o_ref[...] = acc[...] + b_ref[...]


def _m3(degp, s2, u2, w_t, b_r):
    d_out = w_t.shape[1]
    return pl.pallas_call(
        _m3_body,
        grid=(N_PAD // BN, NCHUNK),
        in_specs=[
            pl.BlockSpec((NC, BN, LANES), lambda i, k: (0, i, 0)),
            pl.BlockSpec((1, BN, DC), lambda i, k: (k, i, 0)),
            pl.BlockSpec((BN, DC), lambda i, k: (i, k)),
            pl.BlockSpec((DC, d_out), lambda i, k: (k, 0)),
            pl.BlockSpec((1, d_out), lambda i, k: (0, 0)),
        ],
        out_specs=pl.BlockSpec((BN, d_out), lambda i, k: (i, 0)),
        out_shape=jax.ShapeDtypeStruct((N_PAD, d_out), jnp.float32),
        scratch_shapes=[pltpu.VMEM((BN, d_out), jnp.float32)],
    )(degp, s2, u2, w_t, b_r)


def kernel(x, edge_index, W1, b1, W2, b2, W3, b3):
    n, _ = x.shape
    e = edge_index.shape[1]
    x_p = jnp.pad(x, ((0, N_PAD - n), (0, 0)))
    row3 = jnp.pad(edge_index[0], (0, E_PAD - e)).reshape(NS, NB, EB)
    col3 = jnp.pad(edge_index[1], (0, E_PAD - e), constant_values=n).reshape(
        NS, NB, EB
    )
    rc = jnp.stack(
        [row3.reshape(NS, NBATCH, SEB), col3.reshape(NS, NBATCH, SEB)],
        axis=2,
    )

    degp = _deg_counts(col3)
    g1, u1 = _m1(degp, x_p, W1.T, b1.reshape(1, -1))
    s1 = _edge_scatter(g1, rc)
    g2, u2 = _m2(degp, s1, u1, W2.T, b2.reshape(1, -1))
    s2 = _edge_scatter(g2, rc)
    y = _m3(degp, s2, u2, W3.T, b3.reshape(1, -1))
    return y[:n]


# speedup vs baseline: 1.4997x; 1.4101x over previous
import functools

import jax
import jax.numpy as jnp
from jax import lax
from jax.experimental import pallas as pl
from jax.experimental.pallas import tpu as pltpu
from jax.experimental.pallas import tpu_sc as plsc

NC = 2
NS = 16
LANES = 16

N_PAD = 10240
E_PAD = 163840
EB = 128
NB = E_PAD // (NS * EB)
DH = 512
DC = 128
NCHUNK = DH // DC
RPT = N_PAD // NS
BN = 512


def _sc_mesh():
    return plsc.VectorSubcoreMesh(
        core_axis_name="c", subcore_axis_name="s", num_cores=NC, num_subcores=NS
    )


def _fill_rows(buf, nrows, width, vec):
    for r in range(nrows):
        for i in range(width // LANES):
            buf[r, pl.ds(i * LANES, LANES)] = vec


def _zero_accum_slice(zbuf, zrows, accum, base, nrows):
    off = 0
    while off < nrows:
        step = min(zrows, nrows - off)
        if step == zrows:
            pltpu.sync_copy(zbuf, accum.at[pl.ds(base + off, step)])
        else:
            pltpu.sync_copy(zbuf.at[pl.ds(0, step)], accum.at[pl.ds(base + off, step)])
        off += step


def _deg_body(col_hbm, out_hbm, colbuf, onesbuf, zbuf, accum):
    c = lax.axis_index("c")
    s = lax.axis_index("s")
    _fill_rows(zbuf, 64, LANES, jnp.zeros((LANES,), jnp.float32))
    _fill_rows(onesbuf, EB, LANES, jnp.ones((LANES,), jnp.float32))
    _zero_accum_slice(zbuf, 64, accum, s * RPT, RPT)
    pltpu.sync_copy(col_hbm.at[s].at[pl.ds(c * (NB // NC), NB // NC)], colbuf)
    plsc.subcore_barrier()

    @pl.loop(0, NB // NC)
    def _(j):
        pltpu.sync_copy(onesbuf, accum.at[colbuf.at[j]], add=True)

    plsc.subcore_barrier()
    pltpu.sync_copy(
        accum.at[pl.ds(s * RPT, RPT)], out_hbm.at[c].at[pl.ds(s * RPT, RPT)]
    )


def _deg_counts(col3):
    k = pl.kernel(
        _deg_body,
        out_type=jax.ShapeDtypeStruct((NC, N_PAD, LANES), jnp.float32),
        mesh=_sc_mesh(),
        scratch_types=[
            pltpu.VMEM((NB // NC, EB), jnp.int32),
            pltpu.VMEM((EB, LANES), jnp.float32),
            pltpu.VMEM((64, LANES), jnp.float32),
            pltpu.VMEM_SHARED((N_PAD, LANES), jnp.float32),
        ],
    )
    return k(col3)


SEB = 64
NBATCH = N_PAD // SEB
NBUF = 4
SGB = 32
NSG = NBATCH // SGB


def _scatter_body(g_hbm, rc_hbm, out_hbm, ibuf, gbuf, zbuf, accum, *sems):
    gsem = sems[0:NBUF]
    ssem = sems[NBUF:2 * NBUF]
    isem = sems[2 * NBUF]
    c = lax.axis_index("c")
    s = lax.axis_index("s")
    _fill_rows(zbuf, 8, DC, jnp.zeros((LANES,), jnp.float32))
    for kk in range(NCHUNK // NC):
        chunk = c * (NCHUNK // NC) + kk
        zd = [
            pltpu.make_async_copy(
                zbuf, accum.at[pl.ds(s * RPT + 8 * t, 8)], gsem[0]
            )
            for t in range(RPT // 8)
        ]
        for d in zd:
            d.start()
        for d in zd:
            d.wait()
        plsc.subcore_barrier()

        @pl.loop(0, NSG)
        def _(sg):
            idx = pltpu.make_async_copy(
                rc_hbm.at[s].at[pl.ds(sg * SGB, SGB)], ibuf, isem
            )
            idx.start()
            idx.wait()

            def gd(t):
                return pltpu.make_async_copy(
                    g_hbm.at[chunk].at[ibuf.at[t, 0]],
                    gbuf.at[t % NBUF],
                    gsem[t % NBUF],
                )

            def sd(t):
                return pltpu.make_async_copy(
                    gbuf.at[t % NBUF], accum.at[ibuf.at[t, 1]], ssem[t % NBUF]
                )

            gd(0).start()
            gd(1).start()
            for t in range(SGB):
                gd(t).wait()
                if t >= 2:
                    sd(t - 2).wait()
                sd(t).start(add=True)
                if t + 2 < SGB:
                    gd(t + 2).start()
            sd(SGB - 2).wait()
            sd(SGB - 1).wait()

        plsc.subcore_barrier()
        pltpu.sync_copy(
            accum.at[pl.ds(s * RPT, RPT)],
            out_hbm.at[chunk].at[pl.ds(s * RPT, RPT)],
        )
        plsc.subcore_barrier()


def _edge_scatter(g, rc):
    k = pl.kernel(
        _scatter_body,
        out_type=jax.ShapeDtypeStruct((NCHUNK, N_PAD, DC), jnp.float32),
        mesh=_sc_mesh(),
        scratch_types=[
            pltpu.VMEM((SGB, 2, SEB), jnp.int32),
            pltpu.VMEM((NBUF, SEB, DC), jnp.float32),
            pltpu.VMEM((8, DC), jnp.float32),
            pltpu.VMEM_SHARED((N_PAD, DC), jnp.float32),
        ] + [pltpu.SemaphoreType.DMA] * (2 * NBUF + 1),
    )
    return k(g, rc)


def _dinv(degp_ref):
    p = degp_ref[...]
    cnt = p[0, :, 0] + p[1, :, 0]
    return lax.rsqrt(cnt + 1.0)


def _m1_body(degp_ref, x_ref, w_ref, b_ref, g_ref, u_ref):
    dinv = _dinv(degp_ref)
    h = (
        jnp.dot(x_ref[...], w_ref[...], preferred_element_type=jnp.float32)
        + b_ref[...]
    )
    u_ref[...] = (dinv * dinv)[:, None] * h
    gall = dinv[:, None] * h
    for k in range(NCHUNK):
        g_ref[k] = gall[:, k * DC:(k + 1) * DC]


def _m1(degp, x_p, w_t, b_r):
    d_in = x_p.shape[1]
    return pl.pallas_call(
        _m1_body,
        grid=(N_PAD // BN,),
        in_specs=[
            pl.BlockSpec((NC, BN, LANES), lambda i: (0, i, 0)),
            pl.BlockSpec((BN, d_in), lambda i: (i, 0)),
            pl.BlockSpec((d_in, DH), lambda i: (0, 0)),
            pl.BlockSpec((1, DH), lambda i: (0, 0)),
        ],
        out_specs=[
            pl.BlockSpec((NCHUNK, BN, DC), lambda i: (0, i, 0)),
            pl.BlockSpec((BN, DH), lambda i: (i, 0)),
        ],
        out_shape=[
            jax.ShapeDtypeStruct((NCHUNK, N_PAD, DC), jnp.float32),
            jax.ShapeDtypeStruct((N_PAD, DH), jnp.float32),
        ],
    )(degp, x_p, w_t, b_r)


def _z_block(degp_ref, s_ref, u_ref):
    dinv = _dinv(degp_ref)
    d2 = dinv * dinv
    z = jnp.concatenate([s_ref[k] for k in range(NCHUNK)], axis=1)
    z = jnp.maximum(dinv[:, None] * z + d2[:, None] * u_ref[...], 0.0)
    return z, dinv, d2


def _m2_body(degp_ref, s_ref, u_ref, w_ref, b_ref, g_ref, u2_ref):
    z, dinv, d2 = _z_block(degp_ref, s_ref, u_ref)
    h = jnp.dot(z, w_ref[...], preferred_element_type=jnp.float32) + b_ref[...]
    u2_ref[...] = d2[:, None] * h
    gall = dinv[:, None] * h
    for k in range(NCHUNK):
        g_ref[k] = gall[:, k * DC:(k + 1) * DC]


def _m2(degp, s1, u1, w_t, b_r):
    return pl.pallas_call(
        _m2_body,
        grid=(N_PAD // BN,),
        in_specs=[
            pl.BlockSpec((NC, BN, LANES), lambda i: (0, i, 0)),
            pl.BlockSpec((NCHUNK, BN, DC), lambda i: (0, i, 0)),
            pl.BlockSpec((BN, DH), lambda i: (i, 0)),
            pl.BlockSpec((DH, DH), lambda i: (0, 0)),
            pl.BlockSpec((1, DH), lambda i: (0, 0)),
        ],
        out_specs=[
            pl.BlockSpec((NCHUNK, BN, DC), lambda i: (0, i, 0)),
            pl.BlockSpec((BN, DH), lambda i: (i, 0)),
        ],
        out_shape=[
            jax.ShapeDtypeStruct((NCHUNK, N_PAD, DC), jnp.float32),
            jax.ShapeDtypeStruct((N_PAD, DH), jnp.float32),
        ],
    )(degp, s1, u1, w_t, b_r)


def _m3_body(degp_ref, s_ref, u_ref, w_ref, b_ref, o_ref):
    z, _, _ = _z_block(degp_ref, s_ref, u_ref)
    o_ref[...] = (
        jnp.dot(z, w_ref[...], preferred_element_type=jnp.float32) + b_ref[...]
    )


def _m3(degp, s2, u2, w_t, b_r):
    d_out = w_t.shape[1]
    return pl.pallas_call(
        _m3_body,
        grid=(N_PAD // BN,),
        in_specs=[
            pl.BlockSpec((NC, BN, LANES), lambda i: (0, i, 0)),
            pl.BlockSpec((NCHUNK, BN, DC), lambda i: (0, i, 0)),
            pl.BlockSpec((BN, DH), lambda i: (i, 0)),
            pl.BlockSpec((DH, d_out), lambda i: (0, 0)),
            pl.BlockSpec((1, d_out), lambda i: (0, 0)),
        ],
        out_specs=pl.BlockSpec((BN, d_out), lambda i: (i, 0)),
        out_shape=jax.ShapeDtypeStruct((N_PAD, d_out), jnp.float32),
    )(degp, s2, u2, w_t, b_r)


def kernel(x, edge_index, W1, b1, W2, b2, W3, b3):
    n, _ = x.shape
    e = edge_index.shape[1]
    x_p = jnp.pad(x, ((0, N_PAD - n), (0, 0)))
    row3 = jnp.pad(edge_index[0], (0, E_PAD - e)).reshape(NS, NB, EB)
    col3 = jnp.pad(edge_index[1], (0, E_PAD - e), constant_values=n).reshape(
        NS, NB, EB
    )
    rc = jnp.stack(
        [row3.reshape(NS, NBATCH, SEB), col3.reshape(NS, NBATCH, SEB)],
        axis=2,
    )

    degp = _deg_counts(col3)
    g1, u1 = _m1(degp, x_p, W1.T, b1.reshape(1, -1))
    s1 = _edge_scatter(g1, rc)
    g2, u2 = _m2(degp, s1, u1, W2.T, b2.reshape(1, -1))
    s2 = _edge_scatter(g2, rc)
    y = _m3(degp, s2, u2, W3.T, b3.reshape(1, -1))
    return y[:n]
